# Initial kernel scaffold; baseline (speedup 1.0000x reference)
import jax
import jax.numpy as jnp
from jax.experimental import pallas as pl


def kernel(input, target, W, b, noise, noise_samples):
    raise NotImplementedError("write your pallas kernel here")


# speedup vs baseline: 361.6921x; 361.6921x over previous
import functools

import jax
import jax.numpy as jnp
from jax import lax
from jax.experimental import pallas as pl
from jax.experimental.pallas import tpu as pltpu
from jax.experimental.pallas import tpu_sc as plsc

NTOK = 1000
D = 128
NR = 50
NRP = 64
NORM = 9.0
B = 16384

NC, NS = 2, 16
NW = NC * NS
RPW = B // NW
CH = 128
NCH = RPW // CH

BNW = 8

BLK = 2048
GSTEPS = B // BLK


def _sc_gather(W, b, noise, target, ns_pad):
    mesh = plsc.VectorSubcoreMesh(
        core_axis_name="c", subcore_axis_name="s", num_cores=NC, num_subcores=NS
    )

    @functools.partial(
        pl.kernel,
        out_type=(
            jax.ShapeDtypeStruct((B, D), jnp.float32),
            jax.ShapeDtypeStruct((B, BNW), jnp.float32),
            jax.ShapeDtypeStruct((NRP, D), jnp.float32),
            jax.ShapeDtypeStruct((8, NRP), jnp.float32),
        ),
        mesh=mesh,
        compiler_params=pltpu.CompilerParams(needs_layout_passes=False),
        scratch_types=[
            pltpu.VMEM((NCH, CH), jnp.int32),
            pltpu.VMEM((CH, D), jnp.float32),
            pltpu.VMEM((CH, D), jnp.float32),
            pltpu.VMEM((RPW, BNW), jnp.float32),
            pltpu.VMEM((NRP,), jnp.int32),
            pltpu.VMEM((NRP, D), jnp.float32),
            pltpu.VMEM((8, NRP), jnp.float32),
            pltpu.VMEM((NTOK,), jnp.float32),
            pltpu.VMEM((NTOK,), jnp.float32),
            pltpu.SemaphoreType.DMA,
            pltpu.SemaphoreType.DMA,
        ],
    )
    def k(w_hbm, b_hbm, nz_hbm, t_hbm, ns_hbm,
          wr_hbm, bnt_hbm, wn_hbm, aux_hbm,
          idx_v, rows_a, rows_b, bnt_v, nsi_v, wn_v, aux_v, btab_v, ntab_v,
          sem, semo):
        wid = lax.axis_index("s") * NC + lax.axis_index("c")
        base = wid * RPW
        pltpu.sync_copy(b_hbm, btab_v)
        pltpu.sync_copy(nz_hbm, ntab_v)
        for g in range(NCH):
            pltpu.sync_copy(t_hbm.at[pl.ds(base + g * CH, CH)], idx_v.at[g])
        bufs = [rows_a, rows_b]
        for g in range(NCH):
            i = g % 2
            pltpu.async_copy(w_hbm.at[idx_v.at[g]], bufs[i], sem).wait()
            pltpu.async_copy(
                bufs[i], wr_hbm.at[pl.ds(base + g * CH, CH)], semo).wait()
        lane = lax.iota(jnp.int32, 16)
        col0 = jnp.zeros((16,), jnp.int32)
        col1 = col0 + 1
        for j in range(RPW // 16):
            g, o = divmod(j, CH // 16)
            tv = idx_v[g, pl.ds(o * 16, 16)]
            bv = plsc.load_gather(btab_v, [tv])
            nv = plsc.load_gather(ntab_v, [tv])
            row = j * 16 + lane
            plsc.store_scatter(bnt_v, [row, col0], bv)
            plsc.store_scatter(bnt_v, [row, col1], nv)
        pltpu.sync_copy(bnt_v, bnt_hbm.at[pl.ds(base, RPW)])

        @pl.when(wid == 0)
        def _():
            pltpu.sync_copy(ns_hbm, nsi_v)
            pltpu.async_copy(w_hbm.at[nsi_v], wn_v, sem).wait()
            row0 = jnp.zeros((16,), jnp.int32)
            row1 = row0 + 1
            for j in range(NRP // 16):
                tv = nsi_v[pl.ds(j * 16, 16)]
                bv = plsc.load_gather(btab_v, [tv])
                nv = plsc.load_gather(ntab_v, [tv])
                col = j * 16 + lane
                plsc.store_scatter(aux_v, [row0, col], bv)
                plsc.store_scatter(aux_v, [row1, col], nv)
            pltpu.sync_copy(wn_v, wn_hbm)
            pltpu.sync_copy(aux_v, aux_hbm)

    return k(W, b, noise, target, ns_pad)


def _tc_body(x_ref, wr_ref, bnt_ref, wn_ref, aux_ref, out_ref):
    i = pl.program_id(0)
    x = x_ref[...]
    wr = wr_ref[...]
    bt = bnt_ref[:, 0:1]
    nt = bnt_ref[:, 1:2]
    dlog = jnp.sum(x * wr, axis=1, keepdims=True) + bt - NORM
    dp = jnp.exp(dlog)
    rnn = dlog - jnp.log(dp + NR * nt)

    bn_row = aux_ref[0:1, :]
    nz_row = aux_ref[1:2, :]
    nlog = lax.dot_general(
        x, wn_ref[...], (((1,), (1,)), ((), ())),
        precision=lax.Precision.HIGHEST,
        preferred_element_type=jnp.float32) + bn_row - NORM
    npb = jnp.exp(nlog)
    kn = NR * nz_row
    mask = lax.broadcasted_iota(jnp.int32, (1, NRP), 1) < NR
    terms = jnp.where(mask, jnp.log(kn) - jnp.log(npb + kn), 0.0)
    nloss = jnp.sum(terms, axis=1, keepdims=True)
    tot = jnp.sum(rnn + nloss, axis=0, keepdims=True)

    prev = jnp.where(i == 0, 0.0, out_ref[...])
    out_ref[...] = prev + tot

    @pl.when(i == GSTEPS - 1)
    def _():
        out_ref[...] = out_ref[...] * (-1.0 / B)


def _tc_loss(x, wr, bnt, wn, aux):
    return pl.pallas_call(
        _tc_body,
        grid=(GSTEPS,),
        in_specs=[
            pl.BlockSpec((BLK, D), lambda i: (i, 0)),
            pl.BlockSpec((BLK, D), lambda i: (i, 0)),
            pl.BlockSpec((BLK, BNW), lambda i: (i, 0)),
            pl.BlockSpec((NRP, D), lambda i: (0, 0)),
            pl.BlockSpec((8, NRP), lambda i: (0, 0)),
        ],
        out_specs=pl.BlockSpec((1, 1), lambda i: (0, 0)),
        out_shape=jax.ShapeDtypeStruct((1, 1), jnp.float32),
    )(x, wr, bnt, wn, aux)


def kernel(input, target, W, b, noise, noise_samples):
    target = target.astype(jnp.int32)
    ns_pad = jnp.concatenate(
        [noise_samples.astype(jnp.int32), jnp.zeros((NRP - NR,), jnp.int32)])
    wr, bnt, wn, aux = _sc_gather(W, b, noise, target, ns_pad)
    out = _tc_loss(input, wr, bnt, wn, aux)
    return out[0, 0]
